# baseline (speedup 1.0000x reference)
import jax
import jax.numpy as jnp
from jax import lax
from jax.experimental import pallas as pl
from jax.experimental.pallas import tpu as pltpu
from jax.experimental.pallas import tpu_sc as plsc

L1, L2, N, C = 2048, 4096, 8, 3

QTC = 3072
QSC = L2 - QTC
L2_BLK = 1024
NSC = 32
PARTS = NSC // N
QS = QSC // PARTS
RCH = L1 // 16
QU = 4
BIG = 3.0e38


def _nn_tc_kernel(c1m2_ref, c2_ref, iota_ref, out_ref):
    c1m2 = c1m2_ref[0]
    c2b = c2_ref[0]
    dots2 = jax.lax.dot_general(
        c1m2.astype(jnp.bfloat16), c2b.astype(jnp.bfloat16),
        (((1,), (0,)), ((), ())),
        preferred_element_type=jnp.float32)
    sq1 = 0.25 * jnp.sum(c1m2 * c1m2, axis=1, keepdims=True)
    sq2 = jnp.sum(c2b * c2b, axis=0, keepdims=True)
    d = (sq2 + sq1) + dots2
    dmin = jnp.min(d, axis=0, keepdims=True)
    iota = iota_ref[0]
    idx = jnp.min(jnp.where(d <= dmin, iota, float(L1)), axis=0)
    out_ref[0, 0, :] = idx.astype(jnp.int32)


def _nn_tc(c1t, c2t):
    iota = jnp.arange(L1, dtype=jnp.float32).reshape(1, L1, 1)
    return pl.pallas_call(
        _nn_tc_kernel,
        grid=(N, QTC // L2_BLK),
        in_specs=[
            pl.BlockSpec((1, L1, C), lambda i, j: (i, 0, 0)),
            pl.BlockSpec((1, C, L2_BLK), lambda i, j: (i, 0, j)),
            pl.BlockSpec((1, L1, 1), lambda i, j: (0, 0, 0)),
        ],
        out_specs=pl.BlockSpec((1, 1, L2_BLK), lambda i, j: (i, 0, j)),
        out_shape=jax.ShapeDtypeStruct((N, 1, QTC), jnp.int32),
    )(c1t, c2t, iota)


def _rne_bf16(v):
    t = lax.bitcast_convert_type(v, jnp.int32)
    t = (t + 0x7FFF + (jnp.right_shift(t, 16) & 1)) & jnp.int32(-65536)
    return lax.bitcast_convert_type(t, jnp.float32)


def _sc_body(refs_hbm, qrys_hbm, out_hbm, refraw, qryraw, refv, qryv,
             sq1v, sq2v, outv, sem):
    del sem
    w = lax.axis_index("c") * 16 + lax.axis_index("s")
    n = w // PARTS
    pltpu.sync_copy(refs_hbm.at[n], refraw)
    pltpu.sync_copy(qrys_hbm.at[w], qryraw)

    def sq1_body(cc, _):
        x = refraw[0, pl.ds(cc * 16, 16)]
        y = refraw[1, pl.ds(cc * 16, 16)]
        z = refraw[2, pl.ds(cc * 16, 16)]
        refv[0, pl.ds(cc * 16, 16)] = _rne_bf16(x) * (-2.0)
        refv[1, pl.ds(cc * 16, 16)] = _rne_bf16(y) * (-2.0)
        refv[2, pl.ds(cc * 16, 16)] = _rne_bf16(z) * (-2.0)
        sq1v[pl.ds(cc * 16, 16)] = (x * x + y * y) + z * z
        return 0
    lax.fori_loop(0, RCH, sq1_body, 0)

    def sq2_body(cc, _):
        x = qryraw[0, pl.ds(cc * 16, 16)]
        y = qryraw[1, pl.ds(cc * 16, 16)]
        z = qryraw[2, pl.ds(cc * 16, 16)]
        qryv[0, pl.ds(cc * 16, 16)] = _rne_bf16(x)
        qryv[1, pl.ds(cc * 16, 16)] = _rne_bf16(y)
        qryv[2, pl.ds(cc * 16, 16)] = _rne_bf16(z)
        sq2v[pl.ds(cc * 16, 16)] = (x * x + y * y) + z * z
        return 0
    lax.fori_loop(0, QS // 16, sq2_body, 0)

    iota16 = lax.iota(jnp.int32, 16)

    def lane_min(v):
        for kk in (8, 4, 2, 1):
            v = jnp.minimum(v, v.at[iota16 ^ kk].get(mode="promise_in_bounds"))
        return v

    def group_body(g, _):
        q0 = g * 16
        xqv = qryv[0, pl.ds(q0, 16)]
        yqv = qryv[1, pl.ds(q0, 16)]
        zqv = qryv[2, pl.ds(q0, 16)]
        s2v = sq2v[pl.ds(q0, 16)]
        res = jnp.zeros((16,), jnp.int32)
        for sb in range(16 // QU):
            xq = [xqv[sb * QU + j] for j in range(QU)]
            yq = [yqv[sb * QU + j] for j in range(QU)]
            zq = [zqv[sb * QU + j] for j in range(QU)]
            s2 = [s2v[sb * QU + j] for j in range(QU)]

            def chunk_body(cc, carry):
                rv = list(carry[:QU])
                ri = list(carry[QU:2 * QU])
                idxv = carry[2 * QU]
                xm = refv[0, pl.ds(cc * 16, 16)]
                ym = refv[1, pl.ds(cc * 16, 16)]
                zm = refv[2, pl.ds(cc * 16, 16)]
                s1 = sq1v[pl.ds(cc * 16, 16)]
                for j in range(QU):
                    dots = (xm * xq[j] + ym * yq[j]) + zm * zq[j]
                    d = (s2[j] + s1) + dots
                    cond = d < rv[j]
                    rv[j] = jnp.where(cond, d, rv[j])
                    ri[j] = jnp.where(cond, idxv, ri[j])
                return tuple(rv) + tuple(ri) + (idxv + 16,)

            init = (tuple(jnp.full((16,), BIG, jnp.float32)
                          for _ in range(QU))
                    + tuple(jnp.zeros((16,), jnp.int32) for _ in range(QU))
                    + (iota16,))
            fin = lax.fori_loop(0, RCH, chunk_body, init, unroll=2)
            for j in range(QU):
                rv, ri = fin[j], fin[QU + j]
                mv = lane_min(rv)
                miv = lane_min(jnp.where(rv == mv, ri, jnp.int32(L1)))
                res = jnp.where(iota16 == sb * QU + j, miv, res)
        outv[pl.ds(q0, 16)] = res
        return 0

    lax.fori_loop(0, QS // 16, group_body, 0)
    pltpu.sync_copy(outv, out_hbm.at[w])


def _nn_sc(refs, qrys):
    mesh = plsc.VectorSubcoreMesh(core_axis_name="c", subcore_axis_name="s")
    f = pl.kernel(
        _sc_body, mesh=mesh,
        out_type=jax.ShapeDtypeStruct((NSC, QS), jnp.int32),
        scratch_types=[
            pltpu.VMEM((3, L1), jnp.float32),
            pltpu.VMEM((3, QS), jnp.float32),
            pltpu.VMEM((3, L1), jnp.float32),
            pltpu.VMEM((3, QS), jnp.float32),
            pltpu.VMEM((L1,), jnp.float32),
            pltpu.VMEM((QS,), jnp.float32),
            pltpu.VMEM((QS,), jnp.int32),
            pltpu.SemaphoreType.DMA,
        ],
    )
    return f(refs, qrys)


def kernel(coords1, coords2):
    l1, n, c = coords1.shape
    l2 = coords2.shape[0]
    c1r = jnp.transpose(coords1, (1, 0, 2))
    c2t = jnp.transpose(coords2, (1, 2, 0))

    c1m2 = -2.0 * c1r
    tc_out = _nn_tc(c1m2, c2t[:, :, :QTC])

    refs = jnp.transpose(coords1, (1, 2, 0))
    qrys = c2t[:, :, QTC:].reshape(n, c, PARTS, QS).transpose(0, 2, 1, 3)
    qrys = qrys.reshape(NSC, c, QS)
    sc_out = _nn_sc(refs, qrys)

    full = jnp.concatenate(
        [tc_out.reshape(n, QTC), sc_out.reshape(n, QSC)], axis=1)
    idx0 = full.T.reshape(-1).astype(jnp.int64)
    idx1 = jnp.tile(jnp.arange(n, dtype=jnp.int64), l2)
    return idx0, idx1


# speedup vs baseline: 1.0157x; 1.0157x over previous
import jax
import jax.numpy as jnp
from jax import lax
from jax.experimental import pallas as pl
from jax.experimental.pallas import tpu as pltpu
from jax.experimental.pallas import tpu_sc as plsc

L1, L2, N, C = 2048, 4096, 8, 3

QTC = 3072
QSC = L2 - QTC
L2_BLK = 1024
NSC = 32
PARTS = NSC // N
QS = QSC // PARTS
RCH = L1 // 16
QU = 4
BIG = 3.0e38


def _nn_tc_kernel(c1m2_ref, c2_ref, iota_ref, out_ref):
    c1m2 = c1m2_ref[0]
    c2b = c2_ref[0]
    dots2 = jax.lax.dot_general(
        c1m2.astype(jnp.bfloat16), c2b.astype(jnp.bfloat16),
        (((1,), (0,)), ((), ())),
        preferred_element_type=jnp.float32)
    sq1 = 0.25 * jnp.sum(c1m2 * c1m2, axis=1, keepdims=True)
    sq2 = jnp.sum(c2b * c2b, axis=0, keepdims=True)
    d = (sq2 + sq1) + dots2
    dmin = jnp.min(d, axis=0, keepdims=True)
    iota = iota_ref[0]
    idx = jnp.min(jnp.where(d <= dmin, iota, float(L1)), axis=0)
    out_ref[0, 0, :] = idx.astype(jnp.int32)


def _nn_tc(c1t, c2t):
    iota = jnp.arange(L1, dtype=jnp.float32).reshape(1, L1, 1)
    return pl.pallas_call(
        _nn_tc_kernel,
        grid=(N, QTC // L2_BLK),
        in_specs=[
            pl.BlockSpec((1, L1, C), lambda i, j: (i, 0, 0)),
            pl.BlockSpec((1, C, L2_BLK), lambda i, j: (i, 0, j)),
            pl.BlockSpec((1, L1, 1), lambda i, j: (0, 0, 0)),
        ],
        out_specs=pl.BlockSpec((1, 1, L2_BLK), lambda i, j: (i, 0, j)),
        out_shape=jax.ShapeDtypeStruct((N, 1, QTC), jnp.int32),
    )(c1t, c2t, iota)


def _rne_bf16(v):
    t = lax.bitcast_convert_type(v, jnp.int32)
    t = (t + 0x7FFF + (jnp.right_shift(t, 16) & 1)) & jnp.int32(-65536)
    return lax.bitcast_convert_type(t, jnp.float32)


def _sc_body(refs_hbm, qrys_hbm, out_hbm, refraw, qryraw, refv, qryv,
             sq1v, sq2v, outv, sem):
    del sem
    w = lax.axis_index("c") * 16 + lax.axis_index("s")
    n = w // PARTS
    part = w % PARTS
    pltpu.sync_copy(refs_hbm.at[n], refraw)
    pltpu.sync_copy(qrys_hbm.at[n, :, pl.ds(QTC + part * QS, QS)], qryraw)

    def sq1_body(cc, _):
        x = refraw[0, pl.ds(cc * 16, 16)]
        y = refraw[1, pl.ds(cc * 16, 16)]
        z = refraw[2, pl.ds(cc * 16, 16)]
        refv[0, pl.ds(cc * 16, 16)] = _rne_bf16(x) * (-2.0)
        refv[1, pl.ds(cc * 16, 16)] = _rne_bf16(y) * (-2.0)
        refv[2, pl.ds(cc * 16, 16)] = _rne_bf16(z) * (-2.0)
        sq1v[pl.ds(cc * 16, 16)] = (x * x + y * y) + z * z
        return 0
    lax.fori_loop(0, RCH, sq1_body, 0)

    def sq2_body(cc, _):
        x = qryraw[0, pl.ds(cc * 16, 16)]
        y = qryraw[1, pl.ds(cc * 16, 16)]
        z = qryraw[2, pl.ds(cc * 16, 16)]
        qryv[0, pl.ds(cc * 16, 16)] = _rne_bf16(x)
        qryv[1, pl.ds(cc * 16, 16)] = _rne_bf16(y)
        qryv[2, pl.ds(cc * 16, 16)] = _rne_bf16(z)
        sq2v[pl.ds(cc * 16, 16)] = (x * x + y * y) + z * z
        return 0
    lax.fori_loop(0, QS // 16, sq2_body, 0)

    iota16 = lax.iota(jnp.int32, 16)

    def lane_min(v):
        for kk in (8, 4, 2, 1):
            v = jnp.minimum(v, v.at[iota16 ^ kk].get(mode="promise_in_bounds"))
        return v

    def group_body(g, _):
        q0 = g * 16
        xqv = qryv[0, pl.ds(q0, 16)]
        yqv = qryv[1, pl.ds(q0, 16)]
        zqv = qryv[2, pl.ds(q0, 16)]
        s2v = sq2v[pl.ds(q0, 16)]
        res = jnp.zeros((16,), jnp.int32)
        for sb in range(16 // QU):
            xq = [xqv[sb * QU + j] for j in range(QU)]
            yq = [yqv[sb * QU + j] for j in range(QU)]
            zq = [zqv[sb * QU + j] for j in range(QU)]
            s2 = [s2v[sb * QU + j] for j in range(QU)]

            def chunk_body(cc, carry):
                rv = list(carry[:QU])
                ri = list(carry[QU:2 * QU])
                idxv = carry[2 * QU]
                xm = refv[0, pl.ds(cc * 16, 16)]
                ym = refv[1, pl.ds(cc * 16, 16)]
                zm = refv[2, pl.ds(cc * 16, 16)]
                s1 = sq1v[pl.ds(cc * 16, 16)]
                for j in range(QU):
                    dots = (xm * xq[j] + ym * yq[j]) + zm * zq[j]
                    d = (s2[j] + s1) + dots
                    cond = d < rv[j]
                    rv[j] = jnp.where(cond, d, rv[j])
                    ri[j] = jnp.where(cond, idxv, ri[j])
                return tuple(rv) + tuple(ri) + (idxv + 16,)

            init = (tuple(jnp.full((16,), BIG, jnp.float32)
                          for _ in range(QU))
                    + tuple(jnp.zeros((16,), jnp.int32) for _ in range(QU))
                    + (iota16,))
            fin = lax.fori_loop(0, RCH, chunk_body, init, unroll=2)
            for j in range(QU):
                rv, ri = fin[j], fin[QU + j]
                mv = lane_min(rv)
                miv = lane_min(jnp.where(rv == mv, ri, jnp.int32(L1)))
                res = jnp.where(iota16 == sb * QU + j, miv, res)
        outv[pl.ds(q0, 16)] = res
        return 0

    lax.fori_loop(0, QS // 16, group_body, 0)
    pltpu.sync_copy(outv, out_hbm.at[w])


def _nn_sc(refs, qrys):
    mesh = plsc.VectorSubcoreMesh(core_axis_name="c", subcore_axis_name="s")
    f = pl.kernel(
        _sc_body, mesh=mesh,
        out_type=jax.ShapeDtypeStruct((NSC, QS), jnp.int32),
        scratch_types=[
            pltpu.VMEM((3, L1), jnp.float32),
            pltpu.VMEM((3, QS), jnp.float32),
            pltpu.VMEM((3, L1), jnp.float32),
            pltpu.VMEM((3, QS), jnp.float32),
            pltpu.VMEM((L1,), jnp.float32),
            pltpu.VMEM((QS,), jnp.float32),
            pltpu.VMEM((QS,), jnp.int32),
            pltpu.SemaphoreType.DMA,
        ],
    )
    return f(refs, qrys)


def kernel(coords1, coords2):
    l1, n, c = coords1.shape
    l2 = coords2.shape[0]
    c1r = jnp.transpose(coords1, (1, 0, 2))
    c2t = jnp.transpose(coords2, (1, 2, 0))

    c1m2 = -2.0 * c1r
    tc_out = _nn_tc(c1m2, c2t[:, :, :QTC])

    refs = jnp.transpose(coords1, (1, 2, 0))
    sc_out = _nn_sc(refs, c2t)

    full = jnp.concatenate(
        [tc_out.reshape(n, QTC), sc_out.reshape(n, QSC)], axis=1)
    idx0 = full.T.reshape(-1).astype(jnp.int64)
    idx1 = jnp.tile(jnp.arange(n, dtype=jnp.int64), l2)
    return idx0, idx1
